# baseline (speedup 1.0000x reference)
import functools

import jax
import jax.numpy as jnp
from jax import lax
from jax.experimental import pallas as pl
from jax.experimental.pallas import tpu as pltpu
from jax.experimental.pallas import tpu_sc as plsc

ORIG_VOCAB = 1000000
ITEM_COMPRESSED = 100000
DIM = 64
NUM_TOKENS = 1024 * 200

NC = 2
NS = 16
NW = NC * NS
CHUNK = 128
PER_W = NUM_TOKENS // NW
N_CHUNKS = PER_W // CHUNK


def _body(tok_hbm, ci_hbm, cf_hbm, orig_hbm, item_hbm, out_hbm,
          tok_v, ci_v, cf_v, oidx_v, h0_v, h1_v, wf_v,
          orig_rows, item0_rows, item1_rows,
          ts0, ts1, gs0, gs1, os0, os1):
  wid = (lax.axis_index("s").astype(jnp.int32) * jnp.int32(NC)
         + lax.axis_index("c").astype(jnp.int32))
  wbase = wid * jnp.int32(PER_W)
  tsem = (ts0, ts1)
  gsem = (gs0, gs1)
  osem = (os0, os1)
  NCH = jnp.int32(N_CHUNKS)

  pltpu.sync_copy(ci_hbm, ci_v)
  pltpu.sync_copy(cf_hbm, cf_v)
  au = (ci_v[pl.ds(0, 16)].astype(jnp.uint32), ci_v[pl.ds(48, 16)].astype(jnp.uint32))
  bu = (ci_v[pl.ds(16, 16)].astype(jnp.uint32), ci_v[pl.ds(64, 16)].astype(jnp.uint32))
  pu = (ci_v[pl.ds(32, 16)].astype(jnp.uint32), ci_v[pl.ds(80, 16)].astype(jnp.uint32))
  aop = (cf_v[pl.ds(0, 16)], cf_v[pl.ds(16, 16)])
  mod_c = jnp.uint32(ITEM_COMPRESSED)

  def chunk_off(c):
    return wbase + c * jnp.int32(CHUNK)

  def tok_copy(s, c):
    return pltpu.make_async_copy(
        tok_hbm.at[pl.ds(chunk_off(c), CHUNK)], tok_v.at[jnp.int32(s)], tsem[s])

  def gather_copies(s):
    si = jnp.int32(s)
    return (
        pltpu.make_async_copy(orig_hbm.at[oidx_v.at[si]], orig_rows.at[si], gsem[s]),
        pltpu.make_async_copy(item_hbm.at[h0_v.at[si]], item0_rows.at[si], gsem[s]),
        pltpu.make_async_copy(item_hbm.at[h1_v.at[si]], item1_rows.at[si], gsem[s]),
    )

  def out_copy(s, c):
    return pltpu.make_async_copy(
        orig_rows.at[jnp.int32(s)], out_hbm.at[pl.ds(chunk_off(c), CHUNK), :], osem[s])

  def issue_gathers(s):
    for cp in gather_copies(s):
      cp.start()

  def wait_gathers(s):
    for cp in gather_copies(s):
      cp.wait()

  def hashes(s, c):
    for i in range(CHUNK // 16):
      sl = pl.ds(i * 16, 16)
      t = tok_v[s, sl]
      m = t >= ORIG_VOCAB
      x = jnp.where(m, t - ORIG_VOCAB, 0)
      oidx_v[s, sl] = jnp.where(m, 0, t)
      wf_v[s, sl] = jnp.where(m, jnp.float32(1.0), jnp.float32(0.0))
      xu = x.astype(jnp.uint32)
      xf = x.astype(jnp.float32)
      for j, h_ref in ((0, h0_v), (1, h1_v)):
        q = (xf * aop[j] - 0.02).astype(jnp.int32).astype(jnp.uint32)
        r = xu * au[j] + bu[j] - q * pu[j]
        r = jnp.where(r >= pu[j], r - pu[j], r)
        r = jnp.where(r >= pu[j], r - pu[j], r)
        h_ref[s, sl] = (r % mod_c).astype(jnp.int32)

  def combine(s):
    def group_body(g, _):
      w16 = wf_v[s, pl.ds(g * jnp.int32(16), 16)]
      for l in range(16):
        w = w16[l]
        wo = 1.0 - w
        wi = 0.5 * w
        r = g * jnp.int32(16) + jnp.int32(l)
        for k in range(DIM // 16):
          rs = pl.ds(k * 16, 16)
          orig_rows[s, r, rs] = (orig_rows[s, r, rs] * wo
                                 + (item0_rows[s, r, rs] + item1_rows[s, r, rs]) * wi)
      return 0

    lax.fori_loop(jnp.int32(0), jnp.int32(CHUNK // 16), group_body, 0)

  tok_copy(0, jnp.int32(0)).start()
  tok_copy(0, jnp.int32(0)).wait()
  hashes(0, jnp.int32(0))
  issue_gathers(0)
  tok_copy(1, jnp.int32(1)).start()

  def pair_body(mth, _):
    c0 = mth * jnp.int32(2)
    c1 = c0 + jnp.int32(1)
    c2 = c0 + jnp.int32(2)
    c3 = c0 + jnp.int32(3)
    tok_copy(1, c1).wait()
    hashes(1, c1)

    @pl.when(mth >= jnp.int32(1))
    def _():
      out_copy(1, c1).wait()

    issue_gathers(1)

    @pl.when(c2 < NCH)
    def _():
      tok_copy(0, c2).start()

    wait_gathers(0)
    combine(0)
    out_copy(0, c0).start()

    @pl.when(c2 < NCH)
    def _():
      tok_copy(0, c2).wait()
      hashes(0, c2)
      out_copy(0, c0).wait()
      issue_gathers(0)

    @pl.when(c3 < NCH)
    def _():
      tok_copy(1, c3).start()

    wait_gathers(1)
    combine(1)
    out_copy(1, c1).start()
    return 0

  lax.fori_loop(jnp.int32(0), jnp.int32(N_CHUNKS // 2), pair_body, 0)
  out_copy(0, jnp.int32(0)).wait()
  out_copy(1, jnp.int32(0)).wait()


@functools.partial(jax.jit, static_argnums=())
def _run(tok32, ci, cf, orig_table, item_table):
  mesh = plsc.VectorSubcoreMesh(core_axis_name="c", subcore_axis_name="s")
  k = pl.kernel(
      _body,
      out_type=jax.ShapeDtypeStruct((NUM_TOKENS, DIM), jnp.float32),
      mesh=mesh,
      compiler_params=pltpu.CompilerParams(use_tc_tiling_on_sc=False),
      scratch_types=[
          pltpu.VMEM((2, CHUNK), jnp.int32),
          pltpu.VMEM((96,), jnp.int32),
          pltpu.VMEM((32,), jnp.float32),
          pltpu.VMEM((2, CHUNK), jnp.int32),
          pltpu.VMEM((2, CHUNK), jnp.int32),
          pltpu.VMEM((2, CHUNK), jnp.int32),
          pltpu.VMEM((2, CHUNK), jnp.float32),
          pltpu.VMEM((2, CHUNK, DIM), jnp.float32),
          pltpu.VMEM((2, CHUNK, DIM), jnp.float32),
          pltpu.VMEM((2, CHUNK, DIM), jnp.float32),
          pltpu.SemaphoreType.DMA,
          pltpu.SemaphoreType.DMA,
          pltpu.SemaphoreType.DMA,
          pltpu.SemaphoreType.DMA,
          pltpu.SemaphoreType.DMA,
          pltpu.SemaphoreType.DMA,
      ],
  )
  return k(tok32, ci, cf, orig_table, item_table)


def kernel(input, orig_table, item_table, p, a, b):
  tok32 = input.reshape(-1).astype(jnp.int32)
  a2 = a.reshape(-1)
  b2 = b.reshape(-1)
  p2 = p.reshape(-1)
  ci6 = jnp.stack([a2[0], b2[0], p2[0], a2[1], b2[1], p2[1]]).astype(jnp.int32)
  ci = jnp.broadcast_to(ci6[:, None], (6, 16)).reshape(-1)
  aop = (a2.astype(jnp.float64) / p2.astype(jnp.float64)).astype(jnp.float32)
  cf = jnp.broadcast_to(aop[:, None], (2, 16)).reshape(-1)
  out = _run(tok32, ci, cf, orig_table, item_table)
  return out.reshape(input.shape + (DIM,))


# speedup vs baseline: 3.2956x; 3.2956x over previous
import functools

import jax
import jax.numpy as jnp
from jax import lax
from jax.experimental import pallas as pl
from jax.experimental.pallas import tpu as pltpu
from jax.experimental.pallas import tpu_sc as plsc

ORIG_VOCAB = 1000000
ITEM_COMPRESSED = 100000
DIM = 64
NUM_TOKENS = 1024 * 200

NC = 2
NS = 16
NW = NC * NS
CHUNK = 128
PER_W = NUM_TOKENS // NW
N_CHUNKS = PER_W // CHUNK


def _body(tok_hbm, ci_hbm, cf_hbm, orig_hbm, item_hbm, out_hbm,
          tok_v, ci_v, cf_v, oidx_v, h0_v, h1_v, wf_v,
          orig_rows, item0_rows, item1_rows,
          ts0, ts1, gs0, gs1, os0, os1):
  wid = (lax.axis_index("s").astype(jnp.int32) * jnp.int32(NC)
         + lax.axis_index("c").astype(jnp.int32))
  wbase = wid * jnp.int32(PER_W)
  tsem = (ts0, ts1)
  gsem = (gs0, gs1)
  osem = (os0, os1)
  NCH = jnp.int32(N_CHUNKS)

  pltpu.sync_copy(ci_hbm, ci_v)
  pltpu.sync_copy(cf_hbm, cf_v)
  au = (ci_v[pl.ds(0, 16)].astype(jnp.uint32), ci_v[pl.ds(48, 16)].astype(jnp.uint32))
  bu = (ci_v[pl.ds(16, 16)].astype(jnp.uint32), ci_v[pl.ds(64, 16)].astype(jnp.uint32))
  pu = (ci_v[pl.ds(32, 16)].astype(jnp.uint32), ci_v[pl.ds(80, 16)].astype(jnp.uint32))
  aop = (cf_v[pl.ds(0, 16)], cf_v[pl.ds(16, 16)])
  mod_c = jnp.uint32(ITEM_COMPRESSED)

  def chunk_off(c):
    return wbase + c * jnp.int32(CHUNK)

  def tok_copy(s, c):
    return pltpu.make_async_copy(
        tok_hbm.at[pl.ds(chunk_off(c), CHUNK)], tok_v.at[jnp.int32(s)], tsem[s])

  def gather_copies(s):
    si = jnp.int32(s)
    return (
        pltpu.make_async_copy(orig_hbm.at[oidx_v.at[si]], orig_rows.at[si], gsem[s]),
        pltpu.make_async_copy(item_hbm.at[h0_v.at[si]], item0_rows.at[si], gsem[s]),
        pltpu.make_async_copy(item_hbm.at[h1_v.at[si]], item1_rows.at[si], gsem[s]),
    )

  def out_copy(s, c):
    return pltpu.make_async_copy(
        orig_rows.at[jnp.int32(s)], out_hbm.at[pl.ds(chunk_off(c), CHUNK), :], osem[s])

  def issue_gathers(s):
    for cp in gather_copies(s):
      cp.start()

  def wait_gathers(s):
    for cp in gather_copies(s):
      cp.wait()

  def hashes(s, c):
    for i in range(CHUNK // 16):
      sl = pl.ds(i * 16, 16)
      t = tok_v[s, sl]
      m = t >= ORIG_VOCAB
      x = jnp.where(m, t - ORIG_VOCAB, t)
      oidx_v[s, sl] = x
      wf_v[s, sl] = jnp.where(m, jnp.float32(1.0), jnp.float32(0.0))
      xu = x.astype(jnp.uint32)
      xf = x.astype(jnp.float32)
      for j, h_ref in ((0, h0_v), (1, h1_v)):
        q = (xf * aop[j] - 0.02).astype(jnp.int32).astype(jnp.uint32)
        r = xu * au[j] + bu[j] - q * pu[j]
        r = jnp.where(r >= pu[j], r - pu[j], r)
        r = jnp.where(r >= pu[j], r - pu[j], r)
        h_ref[s, sl] = (r % mod_c).astype(jnp.int32)

  def combine(s):
    def group_body(g, _):
      w16 = wf_v[s, pl.ds(g * jnp.int32(16), 16)]
      for l in range(16):
        w = w16[l]
        wo = 1.0 - w
        wi = 0.5 * w
        r = g * jnp.int32(16) + jnp.int32(l)
        for k in range(DIM // 16):
          rs = pl.ds(k * 16, 16)
          orig_rows[s, r, rs] = (orig_rows[s, r, rs] * wo
                                 + (item0_rows[s, r, rs] + item1_rows[s, r, rs]) * wi)
      return 0

    lax.fori_loop(jnp.int32(0), jnp.int32(CHUNK // 16), group_body, 0)

  tok_copy(0, jnp.int32(0)).start()
  tok_copy(0, jnp.int32(0)).wait()
  hashes(0, jnp.int32(0))
  issue_gathers(0)
  tok_copy(1, jnp.int32(1)).start()

  def pair_body(mth, _):
    c0 = mth * jnp.int32(2)
    c1 = c0 + jnp.int32(1)
    c2 = c0 + jnp.int32(2)
    c3 = c0 + jnp.int32(3)
    tok_copy(1, c1).wait()
    hashes(1, c1)

    @pl.when(mth >= jnp.int32(1))
    def _():
      out_copy(1, c1).wait()

    issue_gathers(1)

    @pl.when(c2 < NCH)
    def _():
      tok_copy(0, c2).start()

    wait_gathers(0)
    combine(0)
    out_copy(0, c0).start()

    @pl.when(c2 < NCH)
    def _():
      tok_copy(0, c2).wait()
      hashes(0, c2)
      out_copy(0, c0).wait()
      issue_gathers(0)

    @pl.when(c3 < NCH)
    def _():
      tok_copy(1, c3).start()

    wait_gathers(1)
    combine(1)
    out_copy(1, c1).start()
    return 0

  lax.fori_loop(jnp.int32(0), jnp.int32(N_CHUNKS // 2), pair_body, 0)
  out_copy(0, jnp.int32(0)).wait()
  out_copy(1, jnp.int32(0)).wait()


@functools.partial(jax.jit, static_argnums=())
def _run(tok32, ci, cf, orig_table, item_table):
  mesh = plsc.VectorSubcoreMesh(core_axis_name="c", subcore_axis_name="s")
  k = pl.kernel(
      _body,
      out_type=jax.ShapeDtypeStruct((NUM_TOKENS, DIM), jnp.float32),
      mesh=mesh,
      compiler_params=pltpu.CompilerParams(use_tc_tiling_on_sc=False),
      scratch_types=[
          pltpu.VMEM((2, CHUNK), jnp.int32),
          pltpu.VMEM((96,), jnp.int32),
          pltpu.VMEM((32,), jnp.float32),
          pltpu.VMEM((2, CHUNK), jnp.int32),
          pltpu.VMEM((2, CHUNK), jnp.int32),
          pltpu.VMEM((2, CHUNK), jnp.int32),
          pltpu.VMEM((2, CHUNK), jnp.float32),
          pltpu.VMEM((2, CHUNK, DIM), jnp.float32),
          pltpu.VMEM((2, CHUNK, DIM), jnp.float32),
          pltpu.VMEM((2, CHUNK, DIM), jnp.float32),
          pltpu.SemaphoreType.DMA,
          pltpu.SemaphoreType.DMA,
          pltpu.SemaphoreType.DMA,
          pltpu.SemaphoreType.DMA,
          pltpu.SemaphoreType.DMA,
          pltpu.SemaphoreType.DMA,
      ],
  )
  return k(tok32, ci, cf, orig_table, item_table)


def kernel(input, orig_table, item_table, p, a, b):
  tok32 = input.reshape(-1).astype(jnp.int32)
  a2 = a.reshape(-1)
  b2 = b.reshape(-1)
  p2 = p.reshape(-1)
  ci6 = jnp.stack([a2[0], b2[0], p2[0], a2[1], b2[1], p2[1]]).astype(jnp.int32)
  ci = jnp.broadcast_to(ci6[:, None], (6, 16)).reshape(-1)
  aop = (a2.astype(jnp.float64) / p2.astype(jnp.float64)).astype(jnp.float32)
  cf = jnp.broadcast_to(aop[:, None], (2, 16)).reshape(-1)
  out = _run(tok32, ci, cf, orig_table, item_table)
  return out.reshape(input.shape + (DIM,))
